# baseline (speedup 1.0000x reference)
import functools

import jax
import jax.numpy as jnp
from jax import lax
from jax.experimental import pallas as pl
from jax.experimental.pallas import tpu as pltpu
from jax.experimental.pallas import tpu_sc as plsc

_ALPHA = 4
_NW = 32
_BATCH = 96


def _make_sc_gather(n_out_rows, row_len, dtype):
    mesh = plsc.VectorSubcoreMesh(core_axis_name="c", subcore_axis_name="s")
    ipw = n_out_rows // _NW
    nb = ipw // _BATCH

    @functools.partial(
        pl.kernel,
        mesh=mesh,
        out_type=jax.ShapeDtypeStruct((n_out_rows, row_len), dtype),
        scratch_types=[
            pltpu.VMEM((_BATCH,), jnp.int32),
            pltpu.VMEM((_BATCH, row_len), dtype),
            pltpu.SemaphoreType.DMA,
        ],
    )
    def k(table_hbm, rows_hbm, out_hbm, idx_v, buf_v, sem):
        wid = lax.axis_index("s") * 2 + lax.axis_index("c")
        base = wid * ipw
        for b in range(nb):
            off = base + b * _BATCH
            pltpu.sync_copy(rows_hbm.at[pl.ds(off, _BATCH)], idx_v)
            pltpu.async_copy(table_hbm.at[idx_v], buf_v, sem).wait()
            pltpu.sync_copy(buf_v, out_hbm.at[pl.ds(off, _BATCH)])

    return k


def kernel(frames):
    C, T, H, W = frames.shape
    n = T // _ALPHA
    idx = jnp.linspace(0, T - 1, n).astype(jnp.int32)
    table = frames.reshape(C * T * H, W)
    g = (jnp.arange(C, dtype=jnp.int32) * T)[:, None] + idx[None, :]
    src_rows = (g.reshape(-1)[:, None] * H
                + jnp.arange(H, dtype=jnp.int32)[None, :]).reshape(-1)
    slow2d = _make_sc_gather(C * n * H, W, frames.dtype)(table, src_rows)
    return (slow2d.reshape(C, n, H, W), frames)


# speedup vs baseline: 1.0936x; 1.0936x over previous
import functools

import jax
import jax.numpy as jnp
from jax import lax
from jax.experimental import pallas as pl
from jax.experimental.pallas import tpu as pltpu
from jax.experimental.pallas import tpu_sc as plsc

_ALPHA = 4
_NW = 32
_BATCH = 96


def _make_sc_gather(n_out_rows, row_len, dtype):
    mesh = plsc.VectorSubcoreMesh(core_axis_name="c", subcore_axis_name="s")
    ipw = n_out_rows // _NW
    nb = ipw // _BATCH

    @functools.partial(
        pl.kernel,
        mesh=mesh,
        out_type=jax.ShapeDtypeStruct((n_out_rows, row_len), dtype),
        scratch_types=[
            pltpu.VMEM((_BATCH,), jnp.int32),
            pltpu.VMEM((_BATCH, row_len), dtype),
            pltpu.SemaphoreType.DMA,
        ],
    )
    def k(table_hbm, rows_hbm, out_hbm, idx_v, buf_v, sem):
        wid = lax.axis_index("s") * 2 + lax.axis_index("c")
        base = wid * ipw
        for b in range(nb):
            off = base + b * _BATCH
            pltpu.sync_copy(rows_hbm.at[pl.ds(off, _BATCH)], idx_v)
            pltpu.async_copy(table_hbm.at[idx_v], buf_v, sem).wait()
            pltpu.sync_copy(buf_v, out_hbm.at[pl.ds(off, _BATCH)])

    return k


def _copy_body(in_ref, out_ref):
    out_ref[...] = in_ref[...]


def _tc_copy(frames):
    C, T, H, W = frames.shape
    tb = 8
    return pl.pallas_call(
        _copy_body,
        grid=(C, T // tb),
        in_specs=[pl.BlockSpec((1, tb, H, W), lambda c, t: (c, t, 0, 0))],
        out_specs=pl.BlockSpec((1, tb, H, W), lambda c, t: (c, t, 0, 0)),
        out_shape=jax.ShapeDtypeStruct((C, T, H, W), frames.dtype),
    )(frames)


def kernel(frames):
    C, T, H, W = frames.shape
    n = T // _ALPHA
    idx = jnp.linspace(0, T - 1, n).astype(jnp.int32)
    table = frames.reshape(C * T * H, W)
    g = (jnp.arange(C, dtype=jnp.int32) * T)[:, None] + idx[None, :]
    src_rows = (g.reshape(-1)[:, None] * H
                + jnp.arange(H, dtype=jnp.int32)[None, :]).reshape(-1)
    slow2d = _make_sc_gather(C * n * H, W, frames.dtype)(table, src_rows)
    return (slow2d.reshape(C, n, H, W), _tc_copy(frames))
